# baseline (speedup 1.0000x reference)
import functools

import jax
import jax.numpy as jnp
from jax import lax
from jax.experimental import pallas as pl
from jax.experimental.pallas import tpu as pltpu
from jax.experimental.pallas import tpu_sc as plsc

B = 16384
EMBED = 512
HIDDEN = 1024
CTX = 1024

NW = 32
ROWS_W = B // NW
R_CH = 16
NCHUNK = ROWS_W // R_CH
RG = 8
L = 16


def _sc_body(e_hbm, h_hbm, c_hbm, wx_hbm, wh_hbm, wc_hbm, b_hbm, out_hbm,
             ebuf, hbuf, cbuf, wxv, whv, wcv, bv, obuf, sem0, sem1):
    wid = lax.axis_index("s") * 2 + lax.axis_index("c")
    base = wid * ROWS_W

    pltpu.sync_copy(wx_hbm, wxv)
    pltpu.sync_copy(wh_hbm, whv)
    pltpu.sync_copy(wc_hbm, wcv)
    pltpu.sync_copy(b_hbm, bv)

    sems = (sem0, sem1)

    def start(g, b):
        row0 = base + g * R_CH
        pltpu.async_copy(e_hbm.at[pl.ds(row0, R_CH)], ebuf.at[b], sems[b])
        pltpu.async_copy(h_hbm.at[pl.ds(row0, R_CH)], hbuf.at[b], sems[b])
        pltpu.async_copy(c_hbm.at[pl.ds(row0, R_CH)], cbuf.at[b], sems[b])

    def wait(g, b):
        row0 = base + g * R_CH
        pltpu.make_async_copy(e_hbm.at[pl.ds(row0, R_CH)], ebuf.at[b], sems[b]).wait()
        pltpu.make_async_copy(h_hbm.at[pl.ds(row0, R_CH)], hbuf.at[b], sems[b]).wait()
        pltpu.make_async_copy(c_hbm.at[pl.ds(row0, R_CH)], cbuf.at[b], sems[b]).wait()

    start(0, 0)
    start(1, 1)

    lane = lax.iota(jnp.int32, L)

    def chunk_body(i, _):
        for b in range(2):
            g = i * 2 + b
            wait(g, b)
            resv = jnp.zeros((L,), jnp.float32)

            def accum(buf, wref, nk, r0, accs):
                def kbody(k, accs):
                    wv = wref[pl.ds(k * L, L)]
                    return tuple(
                        a + buf[b, r0 + j, pl.ds(k * L, L)] * wv
                        for j, a in enumerate(accs)
                    )
                return lax.fori_loop(0, nk, kbody, accs, unroll=4)

            for r0 in range(0, R_CH, RG):
                accs = tuple(jnp.zeros((L,), jnp.float32) for _ in range(RG))
                accs = accum(ebuf, wxv, EMBED // L, r0, accs)
                accs = accum(hbuf, whv, HIDDEN // L, r0, accs)
                accs = accum(cbuf, wcv, CTX // L, r0, accs)
                for j in range(RG):
                    s = jnp.sum(accs[j], axis=0)
                    resv = jnp.where(lane == (r0 + j), s, resv)
            @pl.when(g + 2 < NCHUNK)
            def _():
                start(g + 2, b)
            v = resv + bv[pl.ds(0, L)]
            sig = 1.0 / (1.0 + jnp.exp(-v))
            obuf[b, pl.ds(0, L)] = sig
            row0 = base + g * R_CH
            pltpu.sync_copy(obuf.at[b], out_hbm.at[pl.ds(row0, R_CH)])
        return ()

    lax.fori_loop(0, NCHUNK // 2, chunk_body, ())


@jax.jit
def _gate_sc(embed_t, h_t, context, wx, wh, wc, b16):
    mesh = plsc.VectorSubcoreMesh(core_axis_name="c", subcore_axis_name="s")
    f = functools.partial(
        pl.kernel,
        out_type=jax.ShapeDtypeStruct((B,), jnp.float32),
        mesh=mesh,
        compiler_params=pltpu.CompilerParams(needs_layout_passes=False),
        scratch_types=[
            pltpu.VMEM((2, R_CH, EMBED), jnp.float32),
            pltpu.VMEM((2, R_CH, HIDDEN), jnp.float32),
            pltpu.VMEM((2, R_CH, CTX), jnp.float32),
            pltpu.VMEM((EMBED,), jnp.float32),
            pltpu.VMEM((HIDDEN,), jnp.float32),
            pltpu.VMEM((CTX,), jnp.float32),
            pltpu.VMEM((L,), jnp.float32),
            pltpu.VMEM((2, R_CH), jnp.float32),
            pltpu.SemaphoreType.DMA,
            pltpu.SemaphoreType.DMA,
        ],
    )(_sc_body)
    return f(embed_t, h_t, context, wx, wh, wc, b16)


def kernel(embed_t, h_t, context, W_x, W_h, W_ctx, b_ctx):
    wx = W_x.reshape(EMBED)
    wh = W_h.reshape(HIDDEN)
    wc = W_ctx.reshape(CTX)
    b16 = jnp.broadcast_to(b_ctx, (L,))
    return _gate_sc(embed_t, h_t, context, wx, wh, wc, b16)


# speedup vs baseline: 1.2683x; 1.2683x over previous
import functools

import jax
import jax.numpy as jnp
from jax import lax
from jax.experimental import pallas as pl
from jax.experimental.pallas import tpu as pltpu
from jax.experimental.pallas import tpu_sc as plsc

B = 16384
EMBED = 512
HIDDEN = 1024
CTX = 1024

NW = 32
R_CH = 16
RG = 8
L = 16

S_SC = 6144
ROWS_W = S_SC // NW
NCHUNK = ROWS_W // R_CH

TILE = 1024
TC_OFF = S_SC // TILE


def _sc_body(e_hbm, h_hbm, c_hbm, wx_hbm, wh_hbm, wc_hbm, b_hbm, out_hbm,
             ebuf, hbuf, cbuf, wxv, whv, wcv, bv, obuf, sem0, sem1):
    wid = lax.axis_index("s") * 2 + lax.axis_index("c")
    base = wid * ROWS_W

    pltpu.sync_copy(wx_hbm, wxv)
    pltpu.sync_copy(wh_hbm, whv)
    pltpu.sync_copy(wc_hbm, wcv)
    pltpu.sync_copy(b_hbm, bv)

    sems = (sem0, sem1)

    def start(g, b):
        row0 = base + g * R_CH
        pltpu.async_copy(e_hbm.at[pl.ds(row0, R_CH)], ebuf.at[b], sems[b])
        pltpu.async_copy(h_hbm.at[pl.ds(row0, R_CH)], hbuf.at[b], sems[b])
        pltpu.async_copy(c_hbm.at[pl.ds(row0, R_CH)], cbuf.at[b], sems[b])

    def wait(g, b):
        row0 = base + g * R_CH
        pltpu.make_async_copy(e_hbm.at[pl.ds(row0, R_CH)], ebuf.at[b], sems[b]).wait()
        pltpu.make_async_copy(h_hbm.at[pl.ds(row0, R_CH)], hbuf.at[b], sems[b]).wait()
        pltpu.make_async_copy(c_hbm.at[pl.ds(row0, R_CH)], cbuf.at[b], sems[b]).wait()

    start(0, 0)
    start(1, 1)

    lane = lax.iota(jnp.int32, L)

    def chunk_body(i, _):
        for b in range(2):
            g = i * 2 + b
            wait(g, b)
            resv = jnp.zeros((L,), jnp.float32)

            def accum(buf, wref, nk, r0, accs):
                def kbody(k, accs):
                    wv = wref[pl.ds(k * L, L)]
                    return tuple(
                        a + buf[b, r0 + j, pl.ds(k * L, L)] * wv
                        for j, a in enumerate(accs)
                    )
                return lax.fori_loop(0, nk, kbody, accs, unroll=4)

            for r0 in range(0, R_CH, RG):
                accs = tuple(jnp.zeros((L,), jnp.float32) for _ in range(RG))
                accs = accum(ebuf, wxv, EMBED // L, r0, accs)
                accs = accum(hbuf, whv, HIDDEN // L, r0, accs)
                accs = accum(cbuf, wcv, CTX // L, r0, accs)
                for j in range(RG):
                    s = jnp.sum(accs[j], axis=0)
                    resv = jnp.where(lane == (r0 + j), s, resv)

            @pl.when(g + 2 < NCHUNK)
            def _():
                start(g + 2, b)
            v = resv + bv[pl.ds(0, L)]
            sig = 1.0 / (1.0 + jnp.exp(-v))
            obuf[b, pl.ds(0, L)] = sig
            row0 = base + g * R_CH
            pltpu.sync_copy(obuf.at[b], out_hbm.at[pl.ds(row0, R_CH)])
        return ()

    lax.fori_loop(0, NCHUNK // 2, chunk_body, ())


def _tc_body(e_ref, h_ref, c_ref, wx_ref, wh_ref, wc_ref, b_ref, o_ref):
    s = jnp.sum(e_ref[...] * wx_ref[...], axis=1)
    s = s + jnp.sum(h_ref[...] * wh_ref[...], axis=1)
    s = s + jnp.sum(c_ref[...] * wc_ref[...], axis=1)
    o_ref[...] = jax.nn.sigmoid(s + b_ref[0, 0])


@jax.jit
def _gate_hybrid(embed_t, h_t, context, W_x, W_h, W_ctx, b_ctx):
    wx = W_x.reshape(EMBED)
    wh = W_h.reshape(HIDDEN)
    wc = W_ctx.reshape(CTX)
    b16 = jnp.broadcast_to(b_ctx, (L,))
    b2 = b_ctx.reshape(1, 1)

    mesh = plsc.VectorSubcoreMesh(core_axis_name="c", subcore_axis_name="s")
    sc_f = functools.partial(
        pl.kernel,
        out_type=jax.ShapeDtypeStruct((S_SC,), jnp.float32),
        mesh=mesh,
        compiler_params=pltpu.CompilerParams(needs_layout_passes=False),
        scratch_types=[
            pltpu.VMEM((2, R_CH, EMBED), jnp.float32),
            pltpu.VMEM((2, R_CH, HIDDEN), jnp.float32),
            pltpu.VMEM((2, R_CH, CTX), jnp.float32),
            pltpu.VMEM((EMBED,), jnp.float32),
            pltpu.VMEM((HIDDEN,), jnp.float32),
            pltpu.VMEM((CTX,), jnp.float32),
            pltpu.VMEM((L,), jnp.float32),
            pltpu.VMEM((2, R_CH), jnp.float32),
            pltpu.SemaphoreType.DMA,
            pltpu.SemaphoreType.DMA,
        ],
    )(_sc_body)
    out_sc = sc_f(embed_t, h_t, context, wx, wh, wc, b16)

    grid = ((B - S_SC) // TILE,)
    out_tc = pl.pallas_call(
        _tc_body,
        grid=grid,
        in_specs=[
            pl.BlockSpec((TILE, EMBED), lambda i: (i + TC_OFF, 0)),
            pl.BlockSpec((TILE, HIDDEN), lambda i: (i + TC_OFF, 0)),
            pl.BlockSpec((TILE, CTX), lambda i: (i + TC_OFF, 0)),
            pl.BlockSpec((1, EMBED), lambda i: (0, 0)),
            pl.BlockSpec((1, HIDDEN), lambda i: (0, 0)),
            pl.BlockSpec((1, CTX), lambda i: (0, 0)),
            pl.BlockSpec((1, 1), lambda i: (0, 0)),
        ],
        out_specs=pl.BlockSpec((TILE,), lambda i: (i,)),
        out_shape=jax.ShapeDtypeStruct((B - S_SC,), jnp.float32),
        compiler_params=pltpu.CompilerParams(
            dimension_semantics=("arbitrary",),
        ),
    )(embed_t, h_t, context, W_x, W_h, W_ctx, b2)

    return jnp.concatenate([out_sc, out_tc])


def kernel(embed_t, h_t, context, W_x, W_h, W_ctx, b_ctx):
    return _gate_hybrid(embed_t, h_t, context, W_x, W_h, W_ctx, b_ctx)


# speedup vs baseline: 1.3034x; 1.0277x over previous
import functools

import jax
import jax.numpy as jnp
from jax import lax
from jax.experimental import pallas as pl
from jax.experimental.pallas import tpu as pltpu
from jax.experimental.pallas import tpu_sc as plsc

B = 16384
EMBED = 512
HIDDEN = 1024
CTX = 1024

NW = 32
R_CH = 16
RG = 8
L = 16

S_SC = 2048
ROWS_W = S_SC // NW
NCHUNK = ROWS_W // R_CH

TILE = 1024
TC_OFF = S_SC // TILE


def _sc_body(e_hbm, h_hbm, c_hbm, wx_hbm, wh_hbm, wc_hbm, b_hbm, out_hbm,
             ebuf, hbuf, cbuf, wxv, whv, wcv, bv, obuf, sem0, sem1):
    wid = lax.axis_index("s") * 2 + lax.axis_index("c")
    base = wid * ROWS_W

    pltpu.sync_copy(wx_hbm, wxv)
    pltpu.sync_copy(wh_hbm, whv)
    pltpu.sync_copy(wc_hbm, wcv)
    pltpu.sync_copy(b_hbm, bv)

    sems = (sem0, sem1)

    def start(g, b):
        row0 = base + g * R_CH
        pltpu.async_copy(e_hbm.at[pl.ds(row0, R_CH)], ebuf.at[b], sems[b])
        pltpu.async_copy(h_hbm.at[pl.ds(row0, R_CH)], hbuf.at[b], sems[b])
        pltpu.async_copy(c_hbm.at[pl.ds(row0, R_CH)], cbuf.at[b], sems[b])

    def wait(g, b):
        row0 = base + g * R_CH
        pltpu.make_async_copy(e_hbm.at[pl.ds(row0, R_CH)], ebuf.at[b], sems[b]).wait()
        pltpu.make_async_copy(h_hbm.at[pl.ds(row0, R_CH)], hbuf.at[b], sems[b]).wait()
        pltpu.make_async_copy(c_hbm.at[pl.ds(row0, R_CH)], cbuf.at[b], sems[b]).wait()

    start(0, 0)
    start(1, 1)

    lane = lax.iota(jnp.int32, L)

    def chunk_body(i, _):
        for b in range(2):
            g = i * 2 + b
            wait(g, b)
            resv = jnp.zeros((L,), jnp.float32)

            def accum(buf, wref, nk, r0, accs):
                def kbody(k, accs):
                    wv = wref[pl.ds(k * L, L)]
                    return tuple(
                        a + buf[b, r0 + j, pl.ds(k * L, L)] * wv
                        for j, a in enumerate(accs)
                    )
                return lax.fori_loop(0, nk, kbody, accs, unroll=4)

            for r0 in range(0, R_CH, RG):
                accs = tuple(jnp.zeros((L,), jnp.float32) for _ in range(RG))
                accs = accum(ebuf, wxv, EMBED // L, r0, accs)
                accs = accum(hbuf, whv, HIDDEN // L, r0, accs)
                accs = accum(cbuf, wcv, CTX // L, r0, accs)
                for j in range(RG):
                    s = jnp.sum(accs[j], axis=0)
                    resv = jnp.where(lane == (r0 + j), s, resv)

            @pl.when(g + 2 < NCHUNK)
            def _():
                start(g + 2, b)
            v = resv + bv[pl.ds(0, L)]
            sig = 1.0 / (1.0 + jnp.exp(-v))
            obuf[b, pl.ds(0, L)] = sig
            row0 = base + g * R_CH
            pltpu.sync_copy(obuf.at[b], out_hbm.at[pl.ds(row0, R_CH)])
        return ()

    lax.fori_loop(0, NCHUNK // 2, chunk_body, ())


def _tc_body(e_ref, h_ref, c_ref, wx_ref, wh_ref, wc_ref, b_ref, o_ref):
    s = jnp.sum(e_ref[...] * wx_ref[...], axis=1)
    s = s + jnp.sum(h_ref[...] * wh_ref[...], axis=1)
    s = s + jnp.sum(c_ref[...] * wc_ref[...], axis=1)
    o_ref[...] = jax.nn.sigmoid(s + b_ref[0, 0])


@jax.jit
def _gate_hybrid(embed_t, h_t, context, W_x, W_h, W_ctx, b_ctx):
    wx = W_x.reshape(EMBED)
    wh = W_h.reshape(HIDDEN)
    wc = W_ctx.reshape(CTX)
    b16 = jnp.broadcast_to(b_ctx, (L,))
    b2 = b_ctx.reshape(1, 1)

    mesh = plsc.VectorSubcoreMesh(core_axis_name="c", subcore_axis_name="s")
    sc_f = functools.partial(
        pl.kernel,
        out_type=jax.ShapeDtypeStruct((S_SC,), jnp.float32),
        mesh=mesh,
        compiler_params=pltpu.CompilerParams(needs_layout_passes=False),
        scratch_types=[
            pltpu.VMEM((2, R_CH, EMBED), jnp.float32),
            pltpu.VMEM((2, R_CH, HIDDEN), jnp.float32),
            pltpu.VMEM((2, R_CH, CTX), jnp.float32),
            pltpu.VMEM((EMBED,), jnp.float32),
            pltpu.VMEM((HIDDEN,), jnp.float32),
            pltpu.VMEM((CTX,), jnp.float32),
            pltpu.VMEM((L,), jnp.float32),
            pltpu.VMEM((2, R_CH), jnp.float32),
            pltpu.SemaphoreType.DMA,
            pltpu.SemaphoreType.DMA,
        ],
    )(_sc_body)
    out_sc = sc_f(embed_t, h_t, context, wx, wh, wc, b16)

    grid = ((B - S_SC) // TILE,)
    out_tc = pl.pallas_call(
        _tc_body,
        grid=grid,
        in_specs=[
            pl.BlockSpec((TILE, EMBED), lambda i: (i + TC_OFF, 0)),
            pl.BlockSpec((TILE, HIDDEN), lambda i: (i + TC_OFF, 0)),
            pl.BlockSpec((TILE, CTX), lambda i: (i + TC_OFF, 0)),
            pl.BlockSpec((1, EMBED), lambda i: (0, 0)),
            pl.BlockSpec((1, HIDDEN), lambda i: (0, 0)),
            pl.BlockSpec((1, CTX), lambda i: (0, 0)),
            pl.BlockSpec((1, 1), lambda i: (0, 0)),
        ],
        out_specs=pl.BlockSpec((TILE,), lambda i: (i,)),
        out_shape=jax.ShapeDtypeStruct((B - S_SC,), jnp.float32),
        compiler_params=pltpu.CompilerParams(
            dimension_semantics=("arbitrary",),
        ),
    )(embed_t, h_t, context, W_x, W_h, W_ctx, b2)

    return jnp.concatenate([out_sc, out_tc])


def kernel(embed_t, h_t, context, W_x, W_h, W_ctx, b_ctx):
    return _gate_hybrid(embed_t, h_t, context, W_x, W_h, W_ctx, b_ctx)


# speedup vs baseline: 1.3065x; 1.0024x over previous
import functools

import jax
import jax.numpy as jnp
from jax import lax
from jax.experimental import pallas as pl
from jax.experimental.pallas import tpu as pltpu
from jax.experimental.pallas import tpu_sc as plsc

B = 16384
EMBED = 512
HIDDEN = 1024
CTX = 1024

NW = 32
R_CH = 16
RG = 8
L = 16

S_SC = 2048
ROWS_W = S_SC // NW
NCHUNK = ROWS_W // R_CH

TILE = 1024
TC_OFF = S_SC // TILE


def _sc_body(e_hbm, h_hbm, c_hbm, wx_hbm, wh_hbm, wc_hbm, b_hbm, out_hbm,
             ebuf, hbuf, cbuf, wxv, whv, wcv, bv, obuf, sem0, sem1):
    wid = lax.axis_index("s") * 2 + lax.axis_index("c")
    base = wid * ROWS_W

    pltpu.sync_copy(wx_hbm, wxv)
    pltpu.sync_copy(wh_hbm, whv)
    pltpu.sync_copy(wc_hbm, wcv)
    pltpu.sync_copy(b_hbm, bv)

    sems = (sem0, sem1)

    def start(g, b):
        row0 = base + g * R_CH
        pltpu.async_copy(e_hbm.at[pl.ds(row0, R_CH)], ebuf.at[b], sems[b])
        pltpu.async_copy(h_hbm.at[pl.ds(row0, R_CH)], hbuf.at[b], sems[b])
        pltpu.async_copy(c_hbm.at[pl.ds(row0, R_CH)], cbuf.at[b], sems[b])

    def wait(g, b):
        row0 = base + g * R_CH
        pltpu.make_async_copy(e_hbm.at[pl.ds(row0, R_CH)], ebuf.at[b], sems[b]).wait()
        pltpu.make_async_copy(h_hbm.at[pl.ds(row0, R_CH)], hbuf.at[b], sems[b]).wait()
        pltpu.make_async_copy(c_hbm.at[pl.ds(row0, R_CH)], cbuf.at[b], sems[b]).wait()

    start(0, 0)
    start(1, 1)

    lane = lax.iota(jnp.int32, L)

    def chunk_body(i, _):
        for b in range(2):
            g = i * 2 + b
            wait(g, b)
            resv = jnp.zeros((L,), jnp.float32)

            def accum(buf, wref, nk, r0, accs):
                def kbody(k, accs):
                    wv = wref[pl.ds(k * L, L)]
                    return tuple(
                        a + buf[b, r0 + j, pl.ds(k * L, L)] * wv
                        for j, a in enumerate(accs)
                    )
                return lax.fori_loop(0, nk, kbody, accs, unroll=4)

            for r0 in range(0, R_CH, RG):
                accs = tuple(jnp.zeros((L,), jnp.float32) for _ in range(RG))
                accs = accum(ebuf, wxv, EMBED // L, r0, accs)
                accs = accum(hbuf, whv, HIDDEN // L, r0, accs)
                accs = accum(cbuf, wcv, CTX // L, r0, accs)
                for j in range(RG):
                    s = jnp.sum(accs[j], axis=0)
                    resv = jnp.where(lane == (r0 + j), s, resv)

            @pl.when(g + 2 < NCHUNK)
            def _():
                start(g + 2, b)
            v = resv + bv[pl.ds(0, L)]
            sig = 1.0 / (1.0 + jnp.exp(-v))
            obuf[b, pl.ds(0, L)] = sig
            row0 = base + g * R_CH
            pltpu.sync_copy(obuf.at[b], out_hbm.at[pl.ds(row0, R_CH)])
        return ()

    lax.fori_loop(0, NCHUNK // 2, chunk_body, ())


def _tc_body(e_ref, h_ref, c_ref, wx_ref, wh_ref, wc_ref, b_ref, o_ref):
    s = jnp.sum(e_ref[...] * wx_ref[...], axis=1)
    s = s + jnp.sum(h_ref[...] * wh_ref[...], axis=1)
    s = s + jnp.sum(c_ref[...] * wc_ref[...], axis=1)
    o_ref[...] = jax.nn.sigmoid(s + b_ref[0, 0])


@jax.jit
def _gate_hybrid(embed_t, h_t, context, W_x, W_h, W_ctx, b_ctx):
    wx = W_x.reshape(EMBED)
    wh = W_h.reshape(HIDDEN)
    wc = W_ctx.reshape(CTX)
    b16 = jnp.broadcast_to(b_ctx, (L,))
    b2 = b_ctx.reshape(1, 1)

    grid = ((B - S_SC) // TILE,)
    out_tc = pl.pallas_call(
        _tc_body,
        grid=grid,
        in_specs=[
            pl.BlockSpec((TILE, EMBED), lambda i: (i + TC_OFF, 0)),
            pl.BlockSpec((TILE, HIDDEN), lambda i: (i + TC_OFF, 0)),
            pl.BlockSpec((TILE, CTX), lambda i: (i + TC_OFF, 0)),
            pl.BlockSpec((1, EMBED), lambda i: (0, 0)),
            pl.BlockSpec((1, HIDDEN), lambda i: (0, 0)),
            pl.BlockSpec((1, CTX), lambda i: (0, 0)),
            pl.BlockSpec((1, 1), lambda i: (0, 0)),
        ],
        out_specs=pl.BlockSpec((TILE,), lambda i: (i,)),
        out_shape=jax.ShapeDtypeStruct((B - S_SC,), jnp.float32),
        compiler_params=pltpu.CompilerParams(
            dimension_semantics=("arbitrary",),
        ),
    )(embed_t, h_t, context, W_x, W_h, W_ctx, b2)

    mesh = plsc.VectorSubcoreMesh(core_axis_name="c", subcore_axis_name="s")
    sc_f = functools.partial(
        pl.kernel,
        out_type=jax.ShapeDtypeStruct((S_SC,), jnp.float32),
        mesh=mesh,
        compiler_params=pltpu.CompilerParams(needs_layout_passes=False),
        scratch_types=[
            pltpu.VMEM((2, R_CH, EMBED), jnp.float32),
            pltpu.VMEM((2, R_CH, HIDDEN), jnp.float32),
            pltpu.VMEM((2, R_CH, CTX), jnp.float32),
            pltpu.VMEM((EMBED,), jnp.float32),
            pltpu.VMEM((HIDDEN,), jnp.float32),
            pltpu.VMEM((CTX,), jnp.float32),
            pltpu.VMEM((L,), jnp.float32),
            pltpu.VMEM((2, R_CH), jnp.float32),
            pltpu.SemaphoreType.DMA,
            pltpu.SemaphoreType.DMA,
        ],
    )(_sc_body)
    out_sc = sc_f(embed_t, h_t, context, wx, wh, wc, b16)

    return jnp.concatenate([out_sc, out_tc])


def kernel(embed_t, h_t, context, W_x, W_h, W_ctx, b_ctx):
    return _gate_hybrid(embed_t, h_t, context, W_x, W_h, W_ctx, b_ctx)


# speedup vs baseline: 1.3123x; 1.0044x over previous
import functools

import jax
import jax.numpy as jnp
from jax import lax
from jax.experimental import pallas as pl
from jax.experimental.pallas import tpu as pltpu
from jax.experimental.pallas import tpu_sc as plsc

B = 16384
EMBED = 512
HIDDEN = 1024
CTX = 1024

NW = 32
R_CH = 16
RG = 8
L = 16

S_SC = 1024
ROWS_W = S_SC // NW
NCHUNK = ROWS_W // R_CH

TILE = 1024
TC_OFF = S_SC // TILE


def _sc_body(e_hbm, h_hbm, c_hbm, wx_hbm, wh_hbm, wc_hbm, b_hbm, out_hbm,
             ebuf, hbuf, cbuf, wxv, whv, wcv, bv, obuf, sem0, sem1):
    wid = lax.axis_index("s") * 2 + lax.axis_index("c")
    base = wid * ROWS_W

    pltpu.sync_copy(wx_hbm, wxv)
    pltpu.sync_copy(wh_hbm, whv)
    pltpu.sync_copy(wc_hbm, wcv)
    pltpu.sync_copy(b_hbm, bv)

    sems = (sem0, sem1)

    def start(g, b):
        row0 = base + g * R_CH
        pltpu.async_copy(e_hbm.at[pl.ds(row0, R_CH)], ebuf.at[b], sems[b])
        pltpu.async_copy(h_hbm.at[pl.ds(row0, R_CH)], hbuf.at[b], sems[b])
        pltpu.async_copy(c_hbm.at[pl.ds(row0, R_CH)], cbuf.at[b], sems[b])

    def wait(g, b):
        row0 = base + g * R_CH
        pltpu.make_async_copy(e_hbm.at[pl.ds(row0, R_CH)], ebuf.at[b], sems[b]).wait()
        pltpu.make_async_copy(h_hbm.at[pl.ds(row0, R_CH)], hbuf.at[b], sems[b]).wait()
        pltpu.make_async_copy(c_hbm.at[pl.ds(row0, R_CH)], cbuf.at[b], sems[b]).wait()

    start(0, 0)
    start(1, 1)

    lane = lax.iota(jnp.int32, L)

    def chunk_body(i, _):
        for b in range(2):
            g = i * 2 + b
            wait(g, b)
            resv = jnp.zeros((L,), jnp.float32)

            def accum(buf, wref, nk, r0, accs):
                def kbody(k, accs):
                    wv = wref[pl.ds(k * L, L)]
                    return tuple(
                        a + buf[b, r0 + j, pl.ds(k * L, L)] * wv
                        for j, a in enumerate(accs)
                    )
                return lax.fori_loop(0, nk, kbody, accs, unroll=4)

            for r0 in range(0, R_CH, RG):
                accs = tuple(jnp.zeros((L,), jnp.float32) for _ in range(RG))
                accs = accum(ebuf, wxv, EMBED // L, r0, accs)
                accs = accum(hbuf, whv, HIDDEN // L, r0, accs)
                accs = accum(cbuf, wcv, CTX // L, r0, accs)
                for j in range(RG):
                    s = jnp.sum(accs[j], axis=0)
                    resv = jnp.where(lane == (r0 + j), s, resv)

            @pl.when(g + 2 < NCHUNK)
            def _():
                start(g + 2, b)
            v = resv + bv[pl.ds(0, L)]
            sig = 1.0 / (1.0 + jnp.exp(-v))
            obuf[b, pl.ds(0, L)] = sig
            row0 = base + g * R_CH
            pltpu.sync_copy(obuf.at[b], out_hbm.at[pl.ds(row0, R_CH)])
        return ()

    lax.fori_loop(0, NCHUNK // 2, chunk_body, ())


def _tc_body(e_ref, h_ref, c_ref, wx_ref, wh_ref, wc_ref, b_ref, o_ref):
    s = jnp.sum(e_ref[...] * wx_ref[...], axis=1)
    s = s + jnp.sum(h_ref[...] * wh_ref[...], axis=1)
    s = s + jnp.sum(c_ref[...] * wc_ref[...], axis=1)
    o_ref[...] = jax.nn.sigmoid(s + b_ref[0, 0])


@jax.jit
def _gate_hybrid(embed_t, h_t, context, W_x, W_h, W_ctx, b_ctx):
    wx = W_x.reshape(EMBED)
    wh = W_h.reshape(HIDDEN)
    wc = W_ctx.reshape(CTX)
    b16 = jnp.broadcast_to(b_ctx, (L,))
    b2 = b_ctx.reshape(1, 1)

    grid = ((B - S_SC) // TILE,)
    out_tc = pl.pallas_call(
        _tc_body,
        grid=grid,
        in_specs=[
            pl.BlockSpec((TILE, EMBED), lambda i: (i + TC_OFF, 0)),
            pl.BlockSpec((TILE, HIDDEN), lambda i: (i + TC_OFF, 0)),
            pl.BlockSpec((TILE, CTX), lambda i: (i + TC_OFF, 0)),
            pl.BlockSpec((1, EMBED), lambda i: (0, 0)),
            pl.BlockSpec((1, HIDDEN), lambda i: (0, 0)),
            pl.BlockSpec((1, CTX), lambda i: (0, 0)),
            pl.BlockSpec((1, 1), lambda i: (0, 0)),
        ],
        out_specs=pl.BlockSpec((TILE,), lambda i: (i,)),
        out_shape=jax.ShapeDtypeStruct((B - S_SC,), jnp.float32),
        compiler_params=pltpu.CompilerParams(
            dimension_semantics=("arbitrary",),
        ),
    )(embed_t, h_t, context, W_x, W_h, W_ctx, b2)

    mesh = plsc.VectorSubcoreMesh(core_axis_name="c", subcore_axis_name="s")
    sc_f = functools.partial(
        pl.kernel,
        out_type=jax.ShapeDtypeStruct((S_SC,), jnp.float32),
        mesh=mesh,
        compiler_params=pltpu.CompilerParams(needs_layout_passes=False),
        scratch_types=[
            pltpu.VMEM((2, R_CH, EMBED), jnp.float32),
            pltpu.VMEM((2, R_CH, HIDDEN), jnp.float32),
            pltpu.VMEM((2, R_CH, CTX), jnp.float32),
            pltpu.VMEM((EMBED,), jnp.float32),
            pltpu.VMEM((HIDDEN,), jnp.float32),
            pltpu.VMEM((CTX,), jnp.float32),
            pltpu.VMEM((L,), jnp.float32),
            pltpu.VMEM((2, R_CH), jnp.float32),
            pltpu.SemaphoreType.DMA,
            pltpu.SemaphoreType.DMA,
        ],
    )(_sc_body)
    out_sc = sc_f(embed_t, h_t, context, wx, wh, wc, b16)

    return jnp.concatenate([out_sc, out_tc])


def kernel(embed_t, h_t, context, W_x, W_h, W_ctx, b_ctx):
    return _gate_hybrid(embed_t, h_t, context, W_x, W_h, W_ctx, b_ctx)


# speedup vs baseline: 1.7440x; 1.3289x over previous
import jax
import jax.numpy as jnp
from jax.experimental import pallas as pl
from jax.experimental.pallas import tpu as pltpu

B = 16384
EMBED = 512
HIDDEN = 1024
CTX = 1024

TILE = 2048


def _gate_body(e_ref, h_ref, c_ref, wx_ref, wh_ref, wc_ref, b_ref, o_ref):
    s = jnp.sum(e_ref[...] * wx_ref[...], axis=1)
    s = s + jnp.sum(h_ref[...] * wh_ref[...], axis=1)
    s = s + jnp.sum(c_ref[...] * wc_ref[...], axis=1)
    o_ref[...] = jax.nn.sigmoid(s + b_ref[0, 0])


@jax.jit
def _gate_tc(embed_t, h_t, context, W_x, W_h, W_ctx, b2):
    grid = (B // TILE,)
    return pl.pallas_call(
        _gate_body,
        grid=grid,
        in_specs=[
            pl.BlockSpec((TILE, EMBED), lambda i: (i, 0)),
            pl.BlockSpec((TILE, HIDDEN), lambda i: (i, 0)),
            pl.BlockSpec((TILE, CTX), lambda i: (i, 0)),
            pl.BlockSpec((1, EMBED), lambda i: (0, 0)),
            pl.BlockSpec((1, HIDDEN), lambda i: (0, 0)),
            pl.BlockSpec((1, CTX), lambda i: (0, 0)),
            pl.BlockSpec((1, 1), lambda i: (0, 0)),
        ],
        out_specs=pl.BlockSpec((TILE,), lambda i: (i,)),
        out_shape=jax.ShapeDtypeStruct((B,), jnp.float32),
        compiler_params=pltpu.CompilerParams(
            dimension_semantics=("arbitrary",),
        ),
    )(embed_t, h_t, context, W_x, W_h, W_ctx, b2)


def kernel(embed_t, h_t, context, W_x, W_h, W_ctx, b_ctx):
    b2 = b_ctx.reshape(1, 1)
    return _gate_tc(embed_t, h_t, context, W_x, W_h, W_ctx, b2)


# speedup vs baseline: 1.7494x; 1.0031x over previous
import jax
import jax.numpy as jnp
from jax.experimental import pallas as pl
from jax.experimental.pallas import tpu as pltpu

B = 16384
EMBED = 512
HIDDEN = 1024
CTX = 1024

TILE = 512


def _gate_body(e_ref, h_ref, c_ref, wx_ref, wh_ref, wc_ref, b_ref, o_ref):
    s = jnp.sum(e_ref[...] * wx_ref[...], axis=1)
    s = s + jnp.sum(h_ref[...] * wh_ref[...], axis=1)
    s = s + jnp.sum(c_ref[...] * wc_ref[...], axis=1)
    o_ref[...] = jax.nn.sigmoid(s + b_ref[0, 0])


@jax.jit
def _gate_tc(embed_t, h_t, context, W_x, W_h, W_ctx, b2):
    grid = (B // TILE,)
    return pl.pallas_call(
        _gate_body,
        grid=grid,
        in_specs=[
            pl.BlockSpec((TILE, EMBED), lambda i: (i, 0)),
            pl.BlockSpec((TILE, HIDDEN), lambda i: (i, 0)),
            pl.BlockSpec((TILE, CTX), lambda i: (i, 0)),
            pl.BlockSpec((1, EMBED), lambda i: (0, 0)),
            pl.BlockSpec((1, HIDDEN), lambda i: (0, 0)),
            pl.BlockSpec((1, CTX), lambda i: (0, 0)),
            pl.BlockSpec((1, 1), lambda i: (0, 0)),
        ],
        out_specs=pl.BlockSpec((TILE,), lambda i: (i,)),
        out_shape=jax.ShapeDtypeStruct((B,), jnp.float32),
        compiler_params=pltpu.CompilerParams(
            dimension_semantics=("arbitrary",),
        ),
    )(embed_t, h_t, context, W_x, W_h, W_ctx, b2)


def kernel(embed_t, h_t, context, W_x, W_h, W_ctx, b_ctx):
    b2 = b_ctx.reshape(1, 1)
    return _gate_tc(embed_t, h_t, context, W_x, W_h, W_ctx, b2)


# speedup vs baseline: 1.7696x; 1.0116x over previous
import jax
import jax.numpy as jnp
from jax.experimental import pallas as pl
from jax.experimental.pallas import tpu as pltpu

B = 16384
EMBED = 512
HIDDEN = 1024
CTX = 1024

TILE = 1024


def _gate_body(e_ref, h_ref, c_ref, wx_ref, wh_ref, wc_ref, b_ref, o_ref):
    s = jnp.sum(e_ref[...] * wx_ref[...], axis=1)
    s = s + jnp.sum(h_ref[...] * wh_ref[...], axis=1)
    s = s + jnp.sum(c_ref[...] * wc_ref[...], axis=1)
    o_ref[...] = jax.nn.sigmoid(s + b_ref[0, 0])


@jax.jit
def _gate_tc(embed_t, h_t, context, W_x, W_h, W_ctx, b2):
    grid = (B // TILE,)
    return pl.pallas_call(
        _gate_body,
        grid=grid,
        in_specs=[
            pl.BlockSpec((TILE, EMBED), lambda i: (i, 0)),
            pl.BlockSpec((TILE, HIDDEN), lambda i: (i, 0)),
            pl.BlockSpec((TILE, CTX), lambda i: (i, 0)),
            pl.BlockSpec((1, EMBED), lambda i: (0, 0)),
            pl.BlockSpec((1, HIDDEN), lambda i: (0, 0)),
            pl.BlockSpec((1, CTX), lambda i: (0, 0)),
            pl.BlockSpec((1, 1), lambda i: (0, 0)),
        ],
        out_specs=pl.BlockSpec((TILE,), lambda i: (i,)),
        out_shape=jax.ShapeDtypeStruct((B,), jnp.float32),
        compiler_params=pltpu.CompilerParams(
            dimension_semantics=("arbitrary",),
        ),
    )(embed_t, h_t, context, W_x, W_h, W_ctx, b2)


def kernel(embed_t, h_t, context, W_x, W_h, W_ctx, b_ctx):
    b2 = b_ctx.reshape(1, 1)
    return _gate_tc(embed_t, h_t, context, W_x, W_h, W_ctx, b2)
